# baseline (speedup 1.0000x reference)
import dataclasses
import functools

import jax
import jax.numpy as jnp
from jax import lax
from jax.experimental import pallas as pl
from jax.experimental.pallas import tpu as pltpu
from jax.experimental.pallas import tpu_sc as plsc

_N = 10000
_E = 320000
_D = 128
_NEG = 0.2

_L = 16
_NW = 32
_CA = 128
_SUC = 10
_SU = _CA * _SUC
_NSU = _E // _SU
_FSU = _NSU // _NW
_TSU = _NSU - _FSU * _NW
_NP = 10240
_SLAB = _NP // 16
_CB = 64
_BSUC = _SU // _CB

_SUCA = 20
_SUA = _CA * _SUCA
_NSUA = _E // _SUA
_FSUA = _NSUA // _NW
_TSUA = _NSUA - _FSUA * _NW

_RB = 400
_G1 = _N // _RB
_WB = _E // _G1


def _tc1_body(feat_b, fcw, al, ar, w_b, el_o, er_o, sh_o):
    va = lax.dot_general(fcw[...], al[...], (((0,), (0,)), ((), ())),
                         preferred_element_type=jnp.float32)
    vb = lax.dot_general(fcw[...], ar[...], (((0,), (0,)), ((), ())),
                         preferred_element_type=jnp.float32)
    el = lax.dot_general(feat_b[...], va, (((1,), (0,)), ((), ())),
                         preferred_element_type=jnp.float32)
    er = lax.dot_general(feat_b[...], vb, (((1,), (0,)), ((), ())),
                         preferred_element_type=jnp.float32)
    el_o[...] = el
    er_o[...] = er

    se = jnp.max(el) + jnp.max(er)
    se = jnp.where(se > 0, se, _NEG * se)
    sh_o[...] = jnp.concatenate(
        [jnp.full((1, _D), se, jnp.float32),
         jnp.full((1, _D), jnp.max(w_b[...]), jnp.float32)], axis=0)


_tc1 = pl.pallas_call(
    _tc1_body,
    in_specs=[
        pl.BlockSpec((_N, _D), lambda: (0, 0)),
        pl.BlockSpec((_D, _D), lambda: (0, 0)),
        pl.BlockSpec((_D, 1), lambda: (0, 0)),
        pl.BlockSpec((_D, 1), lambda: (0, 0)),
        pl.BlockSpec((1, _E), lambda: (0, 0)),
    ],
    out_specs=[
        pl.BlockSpec((_N, 1), lambda: (0, 0)),
        pl.BlockSpec((_N, 1), lambda: (0, 0)),
        pl.BlockSpec((2, _D), lambda: (0, 0)),
    ],
    out_shape=[
        jax.ShapeDtypeStruct((_N, 1), jnp.float32),
        jax.ShapeDtypeStruct((_N, 1), jnp.float32),
        jax.ShapeDtypeStruct((2, _D), jnp.float32),
    ],
)


def _tc1b_body(feat_b, fcw, fs_o):
    fs_o[...] = lax.dot_general(feat_b[...], fcw[...], (((1,), (1,)), ((), ())),
                                preferred_element_type=jnp.float32)


_tc1b = pl.pallas_call(
    _tc1b_body,
    grid=(_G1,),
    in_specs=[
        pl.BlockSpec((_RB, _D), lambda i: (i, 0)),
        pl.BlockSpec((_D, _D), lambda i: (0, 0)),
    ],
    out_specs=pl.BlockSpec((_RB, _D), lambda i: (i, 0)),
    out_shape=jax.ShapeDtypeStruct((_N, _D), jnp.float32),
)


def _sc_compiler_params():
    cp = pltpu.CompilerParams()
    if "needs_layout_passes" in pltpu.CompilerParams.__dataclass_fields__:
        cp = dataclasses.replace(cp, needs_layout_passes=False)
    return cp


def _make_sc_a():
    mesh = plsc.VectorSubcoreMesh(core_axis_name="c", subcore_axis_name="s")
    out_type = [
        jax.ShapeDtypeStruct((_E,), jnp.float32),
        jax.ShapeDtypeStruct((_E,), jnp.float32),
        jax.ShapeDtypeStruct((2, _NP), jnp.float32),
        jax.ShapeDtypeStruct((2, _NP), jnp.float32),
    ]
    scratch = [
        pltpu.VMEM((_SUCA, _CA), jnp.int32),
        pltpu.VMEM((_SUCA, _CA), jnp.int32),
        pltpu.VMEM((_SUA,), jnp.float32),
        pltpu.VMEM((_SUA,), jnp.float32),
        pltpu.VMEM((_SUA,), jnp.float32),
        pltpu.VMEM((_SUA,), jnp.float32),
        pltpu.VMEM((_SUA,), jnp.float32),
        pltpu.VMEM((_D,), jnp.float32),
        pltpu.VMEM((_D,), jnp.float32),
        pltpu.VMEM((_SLAB,), jnp.float32),
        pltpu.VMEM_SHARED((_NP,), jnp.float32),
        pltpu.VMEM_SHARED((_NP,), jnp.float32),
        pltpu.SemaphoreType.DMA,
        pltpu.SemaphoreType.DMA,
        pltpu.SemaphoreType.DMA,
        pltpu.SemaphoreType.DMA,
    ]

    @functools.partial(pl.kernel, out_type=out_type, mesh=mesh,
                       scratch_types=scratch,
                       compiler_params=_sc_compiler_params())
    def sc_a(src_h, dst_h, w_h, el_h, er_h, sh_h,
             p_h, q_h, de_h, dw_h,
             src2, dst2, wv, elg, erg, pv, qv, shev, shwv, zv,
             de_sh, dw_sh, sem_in, sem_g, sem_o, sem_sc):
        c = lax.axis_index("c")
        s = lax.axis_index("s")
        wid = s * 2 + c
        pltpu.sync_copy(sh_h.at[0], shev)
        pltpu.sync_copy(sh_h.at[1], shwv)

        @pl.loop(0, _SLAB, step=_L)
        def _(j):
            zv[pl.ds(j, _L)] = jnp.zeros((_L,), jnp.float32)

        pltpu.sync_copy(zv, de_sh.at[pl.ds(s * _SLAB, _SLAB)])
        pltpu.sync_copy(zv, dw_sh.at[pl.ds(s * _SLAB, _SLAB)])
        plsc.subcore_barrier()

        def super_chunk(ci):
            base = ci * _SUA
            ds = [pltpu.async_copy(w_h.at[pl.ds(base, _SUA)], wv, sem_in)]
            for j in range(_SUCA):
                sl = pl.ds(base + j * _CA, _CA)
                ds.append(pltpu.async_copy(src_h.at[sl], src2.at[j], sem_in))
                ds.append(pltpu.async_copy(dst_h.at[sl], dst2.at[j], sem_in))
            for d in ds:
                d.wait()
            ds = []
            for j in range(_SUCA):
                sl = pl.ds(j * _CA, _CA)
                ds.append(pltpu.async_copy(el_h.at[src2.at[j]], elg.at[sl], sem_g))
                ds.append(pltpu.async_copy(er_h.at[dst2.at[j]], erg.at[sl], sem_g))
            for d in ds:
                d.wait()
            she = shev[pl.ds(0, _L)]
            shw = shwv[pl.ds(0, _L)]

            @pl.loop(0, _SUA, step=_L)
            def _(j):
                sl = pl.ds(j, _L)
                x = elg[sl] + erg[sl]
                x = jnp.where(x > 0, x, _NEG * x)
                pv[sl] = jnp.exp(x - she)
                qv[sl] = jnp.exp(wv[sl] - shw)

            ds = [
                pltpu.async_copy(pv, p_h.at[pl.ds(base, _SUA)], sem_o),
                pltpu.async_copy(qv, q_h.at[pl.ds(base, _SUA)], sem_o),
            ]
            for j in range(_SUCA):
                sl = pl.ds(j * _CA, _CA)
                ds.append(pltpu.async_copy(pv.at[sl], de_sh.at[dst2.at[j]],
                                           sem_sc, add=True))
                ds.append(pltpu.async_copy(qv.at[sl], dw_sh.at[dst2.at[j]],
                                           sem_sc, add=True))
            for d in ds:
                d.wait()

        @pl.loop(0, _FSUA)
        def _(i):
            super_chunk(wid + i * _NW)

        @pl.when(wid < _TSUA)
        def _():
            super_chunk(_FSUA * _NW + wid)

        plsc.subcore_barrier()
        sl = pl.ds(s * _SLAB, _SLAB)
        pltpu.sync_copy(de_sh.at[sl], de_h.at[c, sl])
        pltpu.sync_copy(dw_sh.at[sl], dw_h.at[c, sl])

    return sc_a


_sc_a = _make_sc_a()


def _make_sc_b():
    mesh = plsc.VectorSubcoreMesh(core_axis_name="c", subcore_axis_name="s")
    out_type = [
        jax.ShapeDtypeStruct((2, _NP, _D), jnp.float32),
        jax.ShapeDtypeStruct((2 * _NP,), jnp.float32),
        jax.ShapeDtypeStruct((2 * _NP,), jnp.float32),
    ]
    scratch = [
        pltpu.VMEM((_BSUC, _CB), jnp.int32),
        pltpu.VMEM((_BSUC, _CB), jnp.int32),
        pltpu.VMEM((_BSUC, _CB), jnp.int32),
        pltpu.VMEM((_SU,), jnp.float32),
        pltpu.VMEM((_SU,), jnp.float32),
        pltpu.VMEM((_SU,), jnp.float32),
        pltpu.VMEM((_SU,), jnp.float32),
        pltpu.VMEM((_SLAB,), jnp.float32),
        pltpu.VMEM((_SLAB,), jnp.float32),
        pltpu.VMEM((_CB, _D), jnp.float32),
        pltpu.VMEM((_CB, _D), jnp.float32),
        pltpu.VMEM((_CB, _D), jnp.float32),
        pltpu.VMEM((_CB, _D), jnp.float32),
        pltpu.VMEM_SHARED((_NP, _D), jnp.float32),
        pltpu.SemaphoreType.DMA,
        pltpu.SemaphoreType.DMA,
        pltpu.SemaphoreType.DMA,
        pltpu.SemaphoreType.DMA,
        pltpu.SemaphoreType.DMA,
        pltpu.SemaphoreType.DMA,
        pltpu.SemaphoreType.DMA,
        pltpu.SemaphoreType.DMA,
        pltpu.SemaphoreType.DMA,
        pltpu.SemaphoreType.DMA,
    ]

    @functools.partial(pl.kernel, out_type=out_type, mesh=mesh,
                       scratch_types=scratch,
                       compiler_params=_sc_compiler_params())
    def sc_b(src_h, dst_h, p_h, q_h, de_h, dw_h, fs_h,
             out_h, ide_h, idw_h,
             src2, dst2, dst2b, pv, qv, idev, idwv, dt0, dt1,
             rows0, rows1, rows2, rows3,
             acc_sh, sem_in, sem_g, sem_r0, sem_r1, sem_r2, sem_r3,
             sem_s0, sem_s1, sem_s2, sem_s3):
        c = lax.axis_index("c")
        s = lax.axis_index("s")
        wid = s * 2 + c
        rows = (rows0, rows1, rows2, rows3)
        sem_r = (sem_r0, sem_r1, sem_r2, sem_r3)
        sem_s = (sem_s0, sem_s1, sem_s2, sem_s3)
        nbuf = 4
        cofs = jnp.full((_L,), c * _NP, jnp.int32)

        slab = pl.ds(s * _SLAB, _SLAB)
        for (part_h, inv_h) in ((de_h, ide_h), (dw_h, idw_h)):
            d0 = pltpu.async_copy(part_h.at[0, slab], dt0, sem_in)
            d1 = pltpu.async_copy(part_h.at[1, slab], dt1, sem_in)
            d0.wait()
            d1.wait()

            @pl.loop(0, _SLAB, step=_L)
            def _(j):
                sl = pl.ds(j, _L)
                dt0[sl] = 0.5 / (dt0[sl] + dt1[sl])

            pltpu.sync_copy(dt0, inv_h.at[pl.ds(c * _NP + s * _SLAB, _SLAB)])

        @pl.loop(0, _CB)
        def _(r):
            for k in range(_D // _L):
                rows0[r, pl.ds(k * _L, _L)] = jnp.zeros((_L,), jnp.float32)

        for t in range(_SLAB // _CB):
            pltpu.sync_copy(rows0, acc_sh.at[pl.ds(s * _SLAB + t * _CB, _CB)])
        plsc.subcore_barrier()

        def super_chunk(ci):
            base = ci * _SU
            ds = [
                pltpu.async_copy(p_h.at[pl.ds(base, _SU)], pv, sem_in),
                pltpu.async_copy(q_h.at[pl.ds(base, _SU)], qv, sem_in),
            ]
            for j in range(_BSUC):
                sl = pl.ds(base + j * _CB, _CB)
                ds.append(pltpu.async_copy(src_h.at[sl], src2.at[j], sem_in))
                ds.append(pltpu.async_copy(dst_h.at[sl], dst2.at[j], sem_in))
            for d in ds:
                d.wait()

            @pl.loop(0, _BSUC)
            def _(r):
                for k in range(_CB // _L):
                    sl = pl.ds(k * _L, _L)
                    dst2b[r, sl] = dst2[r, sl] + cofs

            ds = []
            for j in range(_BSUC):
                sl = pl.ds(j * _CB, _CB)
                ds.append(pltpu.async_copy(ide_h.at[dst2b.at[j]], idev.at[sl],
                                           sem_g))
                ds.append(pltpu.async_copy(idw_h.at[dst2b.at[j]],
                                           idwv.at[sl], sem_g))
            for d in ds:
                d.wait()

            @pl.loop(0, _SU, step=_L)
            def _(j):
                sl = pl.ds(j, _L)
                idev[sl] = pv[sl] * idev[sl] + qv[sl] * idwv[sl]

            gat = [None] * _BSUC
            sca = [None] * _BSUC
            gat[0] = pltpu.async_copy(fs_h.at[src2.at[0]], rows[0], sem_r[0])
            gat[1] = pltpu.async_copy(fs_h.at[src2.at[1]], rows[1], sem_r[1])
            for j in range(_BSUC):
                b = j % nbuf
                gat[j].wait()
                if j + 2 < _BSUC:
                    if j - 2 >= 0:
                        sca[j - 2].wait()
                    nb = (j + 2) % nbuf
                    gat[j + 2] = pltpu.async_copy(fs_h.at[src2.at[j + 2]],
                                                  rows[nb], sem_r[nb])

                @pl.loop(0, _CB, step=2)
                def _(r):
                    base16 = jnp.full((_L,), j * _CB, jnp.int32) + r
                    a0 = plsc.load_gather(idev, [base16])
                    a1 = plsc.load_gather(idev, [base16 + 1])
                    for kk in range(_D // _L):
                        sl = pl.ds(kk * _L, _L)
                        rows[b][r, sl] = rows[b][r, sl] * a0
                    for kk in range(_D // _L):
                        sl = pl.ds(kk * _L, _L)
                        rows[b][r + 1, sl] = rows[b][r + 1, sl] * a1

                sca[j] = pltpu.async_copy(rows[b], acc_sh.at[dst2.at[j]],
                                          sem_s[b], add=True)
            for j in range(max(0, _BSUC - nbuf), _BSUC):
                sca[j].wait()

        @pl.loop(0, _FSU)
        def _(i):
            super_chunk(wid + i * _NW)

        @pl.when(wid < _TSU)
        def _():
            super_chunk(_FSU * _NW + wid)

        plsc.subcore_barrier()
        for t in range(_SLAB // _CA):
            sl = pl.ds(s * _SLAB + t * _CA, _CA)
            pltpu.sync_copy(acc_sh.at[sl], out_h.at[c, sl])

    return sc_b


_sc_b = _make_sc_b()


_OB = 400


def _tc3_body(p_b, b_b, o_b):
    o_b[...] = p_b[0] + p_b[1] + b_b[...]


_tc3 = pl.pallas_call(
    _tc3_body,
    grid=(_N // _OB,),
    in_specs=[
        pl.BlockSpec((2, _OB, _D), lambda i: (0, i, 0)),
        pl.BlockSpec((1, _D), lambda i: (0, 0)),
    ],
    out_specs=pl.BlockSpec((_OB, _D), lambda i: (i, 0)),
    out_shape=jax.ShapeDtypeStruct((_N, _D), jnp.float32),
)


def kernel(feat, edge_index, w, fc_w, attn_l, attn_r, bias):
    src1 = edge_index[0]
    dst1 = edge_index[1]
    al = attn_l.reshape(_D, 1)
    ar = attn_r.reshape(_D, 1)
    w2 = w.reshape(1, _E)

    el, er, sh = _tc1(feat, fc_w, al, ar, w2)
    fs = _tc1b(feat, fc_w)

    p, q, d_e, d_w = _sc_a(src1, dst1, w, el.reshape(_N), er.reshape(_N), sh)

    part, _, _ = _sc_b(src1, dst1, p, q, d_e, d_w, fs)

    res = _tc3(part, bias.reshape(1, _D))
    return res.reshape(_N, 1, _D)


# speedup vs baseline: 1.0051x; 1.0051x over previous
import dataclasses
import functools

import jax
import jax.numpy as jnp
from jax import lax
from jax.experimental import pallas as pl
from jax.experimental.pallas import tpu as pltpu
from jax.experimental.pallas import tpu_sc as plsc

_N = 10000
_E = 320000
_D = 128
_NEG = 0.2

_L = 16
_NW = 32
_CA = 128
_SUC = 10
_SU = _CA * _SUC
_NSU = _E // _SU
_FSU = _NSU // _NW
_TSU = _NSU - _FSU * _NW
_NP = 10240
_SLAB = _NP // 16
_CB = 64
_BSUC = _SU // _CB

_RB = 400
_G1 = _N // _RB
_WB = _E // _G1


def _tc1_body(feat_b, fcw, al, ar, w_b, el_o, er_o, sh_o):
    va = lax.dot_general(fcw[...], al[...], (((0,), (0,)), ((), ())),
                         preferred_element_type=jnp.float32)
    vb = lax.dot_general(fcw[...], ar[...], (((0,), (0,)), ((), ())),
                         preferred_element_type=jnp.float32)
    el = lax.dot_general(feat_b[...], va, (((1,), (0,)), ((), ())),
                         preferred_element_type=jnp.float32)
    er = lax.dot_general(feat_b[...], vb, (((1,), (0,)), ((), ())),
                         preferred_element_type=jnp.float32)
    el_o[...] = el
    er_o[...] = er

    se = jnp.max(el) + jnp.max(er)
    se = jnp.where(se > 0, se, _NEG * se)
    sh_o[...] = jnp.concatenate(
        [jnp.full((1, _D), se, jnp.float32),
         jnp.full((1, _D), jnp.max(w_b[...]), jnp.float32)], axis=0)


_tc1 = pl.pallas_call(
    _tc1_body,
    in_specs=[
        pl.BlockSpec((_N, _D), lambda: (0, 0)),
        pl.BlockSpec((_D, _D), lambda: (0, 0)),
        pl.BlockSpec((_D, 1), lambda: (0, 0)),
        pl.BlockSpec((_D, 1), lambda: (0, 0)),
        pl.BlockSpec((1, _E), lambda: (0, 0)),
    ],
    out_specs=[
        pl.BlockSpec((_N, 1), lambda: (0, 0)),
        pl.BlockSpec((_N, 1), lambda: (0, 0)),
        pl.BlockSpec((2, _D), lambda: (0, 0)),
    ],
    out_shape=[
        jax.ShapeDtypeStruct((_N, 1), jnp.float32),
        jax.ShapeDtypeStruct((_N, 1), jnp.float32),
        jax.ShapeDtypeStruct((2, _D), jnp.float32),
    ],
)


def _tc1b_body(feat_b, fcw, fs_o):
    fs_o[...] = lax.dot_general(feat_b[...], fcw[...], (((1,), (1,)), ((), ())),
                                preferred_element_type=jnp.float32)


_tc1b = pl.pallas_call(
    _tc1b_body,
    grid=(_G1,),
    in_specs=[
        pl.BlockSpec((_RB, _D), lambda i: (i, 0)),
        pl.BlockSpec((_D, _D), lambda i: (0, 0)),
    ],
    out_specs=pl.BlockSpec((_RB, _D), lambda i: (i, 0)),
    out_shape=jax.ShapeDtypeStruct((_N, _D), jnp.float32),
)


def _sc_compiler_params():
    cp = pltpu.CompilerParams()
    if "needs_layout_passes" in pltpu.CompilerParams.__dataclass_fields__:
        cp = dataclasses.replace(cp, needs_layout_passes=False)
    return cp


def _make_sc_a():
    mesh = plsc.VectorSubcoreMesh(core_axis_name="c", subcore_axis_name="s")
    out_type = [
        jax.ShapeDtypeStruct((_E,), jnp.float32),
        jax.ShapeDtypeStruct((_E,), jnp.float32),
        jax.ShapeDtypeStruct((2, _NP), jnp.float32),
        jax.ShapeDtypeStruct((2, _NP), jnp.float32),
    ]
    scratch = [
        pltpu.VMEM((_SUC, _CA), jnp.int32),
        pltpu.VMEM((_SUC, _CA), jnp.int32),
        pltpu.VMEM((_SU,), jnp.float32),
        pltpu.VMEM((_SU,), jnp.float32),
        pltpu.VMEM((_SU,), jnp.float32),
        pltpu.VMEM((_SU,), jnp.float32),
        pltpu.VMEM((_SU,), jnp.float32),
        pltpu.VMEM((_D,), jnp.float32),
        pltpu.VMEM((_D,), jnp.float32),
        pltpu.VMEM((_SLAB,), jnp.float32),
        pltpu.VMEM_SHARED((_NP,), jnp.float32),
        pltpu.VMEM_SHARED((_NP,), jnp.float32),
        pltpu.SemaphoreType.DMA,
        pltpu.SemaphoreType.DMA,
        pltpu.SemaphoreType.DMA,
        pltpu.SemaphoreType.DMA,
    ]

    @functools.partial(pl.kernel, out_type=out_type, mesh=mesh,
                       scratch_types=scratch,
                       compiler_params=_sc_compiler_params())
    def sc_a(src_h, dst_h, w_h, el_h, er_h, sh_h,
             p_h, q_h, de_h, dw_h,
             src2, dst2, wv, elg, erg, pv, qv, shev, shwv, zv,
             de_sh, dw_sh, sem_in, sem_g, sem_o, sem_sc):
        c = lax.axis_index("c")
        s = lax.axis_index("s")
        wid = s * 2 + c
        pltpu.sync_copy(sh_h.at[0], shev)
        pltpu.sync_copy(sh_h.at[1], shwv)

        @pl.loop(0, _SLAB, step=_L)
        def _(j):
            zv[pl.ds(j, _L)] = jnp.zeros((_L,), jnp.float32)

        pltpu.sync_copy(zv, de_sh.at[pl.ds(s * _SLAB, _SLAB)])
        pltpu.sync_copy(zv, dw_sh.at[pl.ds(s * _SLAB, _SLAB)])
        plsc.subcore_barrier()

        def super_chunk(ci):
            base = ci * _SU
            ds = [pltpu.async_copy(w_h.at[pl.ds(base, _SU)], wv, sem_in)]
            for j in range(_SUC):
                sl = pl.ds(base + j * _CA, _CA)
                ds.append(pltpu.async_copy(src_h.at[sl], src2.at[j], sem_in))
                ds.append(pltpu.async_copy(dst_h.at[sl], dst2.at[j], sem_in))
            for d in ds:
                d.wait()
            ds = []
            for j in range(_SUC):
                sl = pl.ds(j * _CA, _CA)
                ds.append(pltpu.async_copy(el_h.at[src2.at[j]], elg.at[sl], sem_g))
                ds.append(pltpu.async_copy(er_h.at[dst2.at[j]], erg.at[sl], sem_g))
            for d in ds:
                d.wait()
            she = shev[pl.ds(0, _L)]
            shw = shwv[pl.ds(0, _L)]

            @pl.loop(0, _SU, step=_L)
            def _(j):
                sl = pl.ds(j, _L)
                x = elg[sl] + erg[sl]
                x = jnp.where(x > 0, x, _NEG * x)
                pv[sl] = jnp.exp(x - she)
                qv[sl] = jnp.exp(wv[sl] - shw)

            ds = [
                pltpu.async_copy(pv, p_h.at[pl.ds(base, _SU)], sem_o),
                pltpu.async_copy(qv, q_h.at[pl.ds(base, _SU)], sem_o),
            ]
            for j in range(_SUC):
                sl = pl.ds(j * _CA, _CA)
                ds.append(pltpu.async_copy(pv.at[sl], de_sh.at[dst2.at[j]],
                                           sem_sc, add=True))
                ds.append(pltpu.async_copy(qv.at[sl], dw_sh.at[dst2.at[j]],
                                           sem_sc, add=True))
            for d in ds:
                d.wait()

        @pl.loop(0, _FSU)
        def _(i):
            super_chunk(wid + i * _NW)

        @pl.when(wid < _TSU)
        def _():
            super_chunk(_FSU * _NW + wid)

        plsc.subcore_barrier()
        sl = pl.ds(s * _SLAB, _SLAB)
        pltpu.sync_copy(de_sh.at[sl], de_h.at[c, sl])
        pltpu.sync_copy(dw_sh.at[sl], dw_h.at[c, sl])

    return sc_a


_sc_a = _make_sc_a()


def _make_sc_b():
    mesh = plsc.VectorSubcoreMesh(core_axis_name="c", subcore_axis_name="s")
    out_type = [
        jax.ShapeDtypeStruct((2, _NP, _D), jnp.float32),
        jax.ShapeDtypeStruct((2 * _NP,), jnp.float32),
        jax.ShapeDtypeStruct((2 * _NP,), jnp.float32),
    ]
    scratch = [
        pltpu.VMEM((_BSUC, _CB), jnp.int32),
        pltpu.VMEM((_BSUC, _CB), jnp.int32),
        pltpu.VMEM((_BSUC, _CB), jnp.int32),
        pltpu.VMEM((_SU,), jnp.float32),
        pltpu.VMEM((_SU,), jnp.float32),
        pltpu.VMEM((_SU,), jnp.float32),
        pltpu.VMEM((_SU,), jnp.float32),
        pltpu.VMEM((_SLAB,), jnp.float32),
        pltpu.VMEM((_SLAB,), jnp.float32),
        pltpu.VMEM((_CB, _D), jnp.float32),
        pltpu.VMEM((_CB, _D), jnp.float32),
        pltpu.VMEM((_CB, _D), jnp.float32),
        pltpu.VMEM((_CB, _D), jnp.float32),
        pltpu.VMEM_SHARED((_NP, _D), jnp.float32),
        pltpu.SemaphoreType.DMA,
        pltpu.SemaphoreType.DMA,
        pltpu.SemaphoreType.DMA,
        pltpu.SemaphoreType.DMA,
        pltpu.SemaphoreType.DMA,
        pltpu.SemaphoreType.DMA,
        pltpu.SemaphoreType.DMA,
        pltpu.SemaphoreType.DMA,
        pltpu.SemaphoreType.DMA,
        pltpu.SemaphoreType.DMA,
    ]

    @functools.partial(pl.kernel, out_type=out_type, mesh=mesh,
                       scratch_types=scratch,
                       compiler_params=_sc_compiler_params())
    def sc_b(src_h, dst_h, p_h, q_h, de_h, dw_h, fs_h,
             out_h, ide_h, idw_h,
             src2, dst2, dst2b, pv, qv, idev, idwv, dt0, dt1,
             rows0, rows1, rows2, rows3,
             acc_sh, sem_in, sem_g, sem_r0, sem_r1, sem_r2, sem_r3,
             sem_s0, sem_s1, sem_s2, sem_s3):
        c = lax.axis_index("c")
        s = lax.axis_index("s")
        wid = s * 2 + c
        rows = (rows0, rows1, rows2, rows3)
        sem_r = (sem_r0, sem_r1, sem_r2, sem_r3)
        sem_s = (sem_s0, sem_s1, sem_s2, sem_s3)
        nbuf = 4
        cofs = jnp.full((_L,), c * _NP, jnp.int32)

        slab = pl.ds(s * _SLAB, _SLAB)
        for (part_h, inv_h) in ((de_h, ide_h), (dw_h, idw_h)):
            d0 = pltpu.async_copy(part_h.at[0, slab], dt0, sem_in)
            d1 = pltpu.async_copy(part_h.at[1, slab], dt1, sem_in)
            d0.wait()
            d1.wait()

            @pl.loop(0, _SLAB, step=_L)
            def _(j):
                sl = pl.ds(j, _L)
                dt0[sl] = 0.5 / (dt0[sl] + dt1[sl])

            pltpu.sync_copy(dt0, inv_h.at[pl.ds(c * _NP + s * _SLAB, _SLAB)])

        @pl.loop(0, _CB)
        def _(r):
            for k in range(_D // _L):
                rows0[r, pl.ds(k * _L, _L)] = jnp.zeros((_L,), jnp.float32)

        for t in range(_SLAB // _CB):
            pltpu.sync_copy(rows0, acc_sh.at[pl.ds(s * _SLAB + t * _CB, _CB)])
        plsc.subcore_barrier()

        def super_chunk(ci):
            base = ci * _SU
            ds = [
                pltpu.async_copy(p_h.at[pl.ds(base, _SU)], pv, sem_in),
                pltpu.async_copy(q_h.at[pl.ds(base, _SU)], qv, sem_in),
            ]
            for j in range(_BSUC):
                sl = pl.ds(base + j * _CB, _CB)
                ds.append(pltpu.async_copy(src_h.at[sl], src2.at[j], sem_in))
                ds.append(pltpu.async_copy(dst_h.at[sl], dst2.at[j], sem_in))
            for d in ds:
                d.wait()

            @pl.loop(0, _BSUC)
            def _(r):
                for k in range(_CB // _L):
                    sl = pl.ds(k * _L, _L)
                    dst2b[r, sl] = dst2[r, sl] + cofs

            ds = []
            for j in range(_BSUC):
                sl = pl.ds(j * _CB, _CB)
                ds.append(pltpu.async_copy(ide_h.at[dst2b.at[j]], idev.at[sl],
                                           sem_g))
                ds.append(pltpu.async_copy(idw_h.at[dst2b.at[j]],
                                           idwv.at[sl], sem_g))
            for d in ds:
                d.wait()

            @pl.loop(0, _SU, step=_L)
            def _(j):
                sl = pl.ds(j, _L)
                idev[sl] = pv[sl] * idev[sl] + qv[sl] * idwv[sl]

            gat = [None] * _BSUC
            sca = [None] * _BSUC
            gat[0] = pltpu.async_copy(fs_h.at[src2.at[0]], rows[0], sem_r[0])
            gat[1] = pltpu.async_copy(fs_h.at[src2.at[1]], rows[1], sem_r[1])
            for j in range(_BSUC):
                b = j % nbuf
                gat[j].wait()
                if j + 2 < _BSUC:
                    if j - 2 >= 0:
                        sca[j - 2].wait()
                    nb = (j + 2) % nbuf
                    gat[j + 2] = pltpu.async_copy(fs_h.at[src2.at[j + 2]],
                                                  rows[nb], sem_r[nb])

                @pl.loop(0, _CB, step=2)
                def _(r):
                    base16 = jnp.full((_L,), j * _CB, jnp.int32) + r
                    a0 = plsc.load_gather(idev, [base16])
                    a1 = plsc.load_gather(idev, [base16 + 1])
                    for kk in range(_D // _L):
                        sl = pl.ds(kk * _L, _L)
                        rows[b][r, sl] = rows[b][r, sl] * a0
                    for kk in range(_D // _L):
                        sl = pl.ds(kk * _L, _L)
                        rows[b][r + 1, sl] = rows[b][r + 1, sl] * a1

                sca[j] = pltpu.async_copy(rows[b], acc_sh.at[dst2.at[j]],
                                          sem_s[b], add=True)
            for j in range(max(0, _BSUC - nbuf), _BSUC):
                sca[j].wait()

        @pl.loop(0, _FSU)
        def _(i):
            super_chunk(wid + i * _NW)

        @pl.when(wid < _TSU)
        def _():
            super_chunk(_FSU * _NW + wid)

        plsc.subcore_barrier()
        for t in range(_SLAB // _CA):
            sl = pl.ds(s * _SLAB + t * _CA, _CA)
            pltpu.sync_copy(acc_sh.at[sl], out_h.at[c, sl])

    return sc_b


_sc_b = _make_sc_b()


_OB = 400


def _tc3_body(p_b, b_b, o_b):
    o_b[...] = p_b[0] + p_b[1] + b_b[...]


_tc3 = pl.pallas_call(
    _tc3_body,
    grid=(_N // _OB,),
    in_specs=[
        pl.BlockSpec((2, _OB, _D), lambda i: (0, i, 0)),
        pl.BlockSpec((1, _D), lambda i: (0, 0)),
    ],
    out_specs=pl.BlockSpec((_OB, _D), lambda i: (i, 0)),
    out_shape=jax.ShapeDtypeStruct((_N, _D), jnp.float32),
)


def kernel(feat, edge_index, w, fc_w, attn_l, attn_r, bias):
    src1 = edge_index[0]
    dst1 = edge_index[1]
    al = attn_l.reshape(_D, 1)
    ar = attn_r.reshape(_D, 1)
    w2 = w.reshape(1, _E)

    el, er, sh = _tc1(feat, fc_w, al, ar, w2)
    fs = _tc1b(feat, fc_w)

    p, q, d_e, d_w = _sc_a(src1, dst1, w, el.reshape(_N), er.reshape(_N), sh)

    part, _, _ = _sc_b(src1, dst1, p, q, d_e, d_w, fs)

    res = _tc3(part, bias.reshape(1, _D))
    return res.reshape(_N, 1, _D)
